# Initial kernel scaffold; baseline (speedup 1.0000x reference)
import jax
import jax.numpy as jnp
from jax.experimental import pallas as pl


def kernel(x):
    raise NotImplementedError("write your pallas kernel here")


# speedup vs baseline: 940.2042x; 940.2042x over previous
import numpy as np
import jax
import jax.numpy as jnp
from jax.experimental import pallas as pl
from jax.experimental.pallas import tpu as pltpu

_DET_LENGTH = 128
_REPS = 3
_N_STATES = 64
_CLIP = 20.0
_T = _REPS * _DET_LENGTH
_GM = np.array([[1, 0, 1, 1, 0, 1, 1], [1, 1, 1, 1, 0, 0, 1]], dtype=np.int64)


def _branch_sign_tables():
    j = np.arange(32)
    reg = np.stack([np.zeros(32, np.int64)] + [(j >> i) & 1 for i in range(6)], axis=1)
    bits = (reg @ _GM.T) % 2
    s0 = 1 - 2 * bits[:, 0]
    s1 = 1 - 2 * bits[:, 1]
    aE = np.where(s0 == s1, s0, 0).astype(np.float32).reshape(32, 1)
    bE = np.where(s0 != s1, s0, 0).astype(np.float32).reshape(32, 1)
    return aE, bE


_AE, _BE = _branch_sign_tables()
_J = np.arange(32)
_WE = (np.uint32(1) << ((2 * _J) % 32).astype(np.uint32)).view(np.int32).reshape(32, 1)
_WO = (np.uint32(1) << ((2 * _J + 1) % 32).astype(np.uint32)).view(np.int32).reshape(32, 1)

_BB = 256
_NG = _T // 8


def _viterbi_body(x0_ref, x1_ref, ae_ref, be_ref, we_ref, wo_ref, out_ref, dec_ref):
    bb = x0_ref.shape[2]
    aE = ae_ref[...]
    bE = be_ref[...]
    wE = we_ref[...]
    wO = wo_ref[...]

    def fwd_group(g, prob):
        pg = jax.lax.rem(g, 16)
        x0 = jnp.clip(x0_ref[pg], -_CLIP, _CLIP).astype(jnp.bfloat16).astype(jnp.float32)
        x1 = jnp.clip(x1_ref[pg], -_CLIP, _CLIP).astype(jnp.bfloat16).astype(jnp.float32)
        sp8 = x0 + x1
        sm8 = x0 - x1
        rows = []
        for i in range(8):
            sp = sp8[i : i + 1, :]
            sm = sm8[i : i + 1, :]
            br = aE * sp + bE * sm
            top = prob[0:32]
            bot = prob[32:64]
            c0e = top + br
            c1e = bot - br
            oe = jnp.maximum(c0e, c1e)
            de = c1e > c0e
            c0o = top - br
            c1o = bot + br
            oo = jnp.maximum(c0o, c1o)
            do = c1o > c0o
            nxt = jnp.stack([oe, oo], axis=1).reshape(64, bb)
            prob = nxt - jnp.max(nxt, axis=0, keepdims=True)
            pw = jnp.where(de, wE, jnp.int32(0)) + jnp.where(do, wO, jnp.int32(0))
            rows.append(jnp.sum(pw[0:16], axis=0, keepdims=True))
            rows.append(jnp.sum(pw[16:32], axis=0, keepdims=True))
        dec_ref[g] = jnp.concatenate(rows, axis=0)
        return prob

    prob0 = jnp.zeros((64, bb), jnp.float32)
    jax.lax.fori_loop(0, _NG, fwd_group, prob0)

    def _step_back(state, w0, w1):
        w = jnp.where(state < 32, w0, w1)
        sh = state & 31
        d = (w >> sh) & 1
        return (state >> 1) + d * 32

    def tb_group(gi, state):
        tile = dec_ref[_NG - 1 - gi]
        for i in range(7, -1, -1):
            state = _step_back(state, tile[2 * i : 2 * i + 1], tile[2 * i + 1 : 2 * i + 2])
        return state

    state = jnp.zeros((1, bb), jnp.int32)
    state = jax.lax.fori_loop(0, 16, tb_group, state)

    def tb_out_group(gi, state):
        g = 31 - gi
        tile = dec_ref[g]
        bits = [None] * 8
        for i in range(7, -1, -1):
            bits[i] = ((state + 1) & 1).astype(jnp.float32)
            state = _step_back(state, tile[2 * i : 2 * i + 1], tile[2 * i + 1 : 2 * i + 2])
        out_ref[g - 16] = jnp.concatenate(bits, axis=0)
        return state

    jax.lax.fori_loop(0, 16, tb_out_group, state)


def kernel(x):
    b = x.shape[0]
    bb = min(_BB, b)
    x0 = jnp.transpose(x[:, 0::2]).reshape(16, 8, b)
    x1 = jnp.transpose(x[:, 1::2]).reshape(16, 8, b)
    res = pl.pallas_call(
        _viterbi_body,
        out_shape=jax.ShapeDtypeStruct((16, 8, b), jnp.float32),
        grid=(b // bb,),
        in_specs=[
            pl.BlockSpec((16, 8, bb), lambda i: (0, 0, i)),
            pl.BlockSpec((16, 8, bb), lambda i: (0, 0, i)),
            pl.BlockSpec((32, 1), lambda i: (0, 0)),
            pl.BlockSpec((32, 1), lambda i: (0, 0)),
            pl.BlockSpec((32, 1), lambda i: (0, 0)),
            pl.BlockSpec((32, 1), lambda i: (0, 0)),
        ],
        out_specs=pl.BlockSpec((16, 8, bb), lambda i: (0, 0, i)),
        scratch_shapes=[pltpu.VMEM((_NG, 16, bb), jnp.int32)],
    )(x0, x1, jnp.asarray(_AE), jnp.asarray(_BE), jnp.asarray(_WE), jnp.asarray(_WO))
    return res.reshape(_DET_LENGTH, b).transpose(1, 0)


# speedup vs baseline: 1198.4762x; 1.2747x over previous
import numpy as np
import jax
import jax.numpy as jnp
from jax.experimental import pallas as pl
from jax.experimental.pallas import tpu as pltpu

_DET_LENGTH = 128
_REPS = 3
_N_STATES = 64
_CLIP = 20.0
_T = _REPS * _DET_LENGTH
_GM = np.array([[1, 0, 1, 1, 0, 1, 1], [1, 1, 1, 1, 0, 0, 1]], dtype=np.int64)


def _branch_sign_tables():
    j = np.arange(32)
    reg = np.stack([np.zeros(32, np.int64)] + [(j >> i) & 1 for i in range(6)], axis=1)
    bits = (reg @ _GM.T) % 2
    s0 = 1 - 2 * bits[:, 0]
    s1 = 1 - 2 * bits[:, 1]
    aE = np.where(s0 == s1, s0, 0).astype(np.float32).reshape(32, 1)
    bE = np.where(s0 != s1, s0, 0).astype(np.float32).reshape(32, 1)
    return aE, bE


_AE, _BE = _branch_sign_tables()
_J = np.arange(32)
_WE = (np.uint32(1) << ((2 * _J) % 32).astype(np.uint32)).view(np.int32).reshape(32, 1)
_WO = (np.uint32(1) << ((2 * _J + 1) % 32).astype(np.uint32)).view(np.int32).reshape(32, 1)

_BB = 512
_NG = _T // 8


def _viterbi_body(x0_ref, x1_ref, ae_ref, be_ref, we_ref, wo_ref, out_ref, dec_ref):
    bb = x0_ref.shape[2]
    aE = jnp.broadcast_to(ae_ref[...], (32, bb))
    bE = jnp.broadcast_to(be_ref[...], (32, bb))
    wE = jnp.broadcast_to(we_ref[...], (32, bb))
    wO = jnp.broadcast_to(wo_ref[...], (32, bb))

    def _llr_rows(pg):
        x0 = jnp.clip(x0_ref[pg], -_CLIP, _CLIP).astype(jnp.bfloat16).astype(jnp.float32)
        x1 = jnp.clip(x1_ref[pg], -_CLIP, _CLIP).astype(jnp.bfloat16).astype(jnp.float32)
        return x0 + x1, x0 - x1

    def _acs(prob, sp, sm):
        br = aE * sp + bE * sm
        top = prob[0:32]
        bot = prob[32:64]
        c0e = top + br
        c1e = bot - br
        oe = jnp.maximum(c0e, c1e)
        c0o = top - br
        c1o = bot + br
        oo = jnp.maximum(c0o, c1o)
        nxt = jnp.stack([oe, oo], axis=1).reshape(64, bb)
        prob = nxt - jnp.max(nxt, axis=0, keepdims=True)
        return prob, (c1e > c0e), (c1o > c0o)

    def fwd_group_nostore(g, prob):
        sp8, sm8 = _llr_rows(g)
        for i in range(8):
            prob, _, _ = _acs(prob, sp8[i : i + 1, :], sm8[i : i + 1, :])
        return prob

    def fwd_group(g, prob):
        sp8, sm8 = _llr_rows(jax.lax.rem(g + 16, 16))
        rows = []
        for i in range(8):
            prob, de, do = _acs(prob, sp8[i : i + 1, :], sm8[i : i + 1, :])
            pw = jnp.where(de, wE, jnp.int32(0)) + jnp.where(do, wO, jnp.int32(0))
            rows.append(jnp.sum(pw[0:16], axis=0, keepdims=True))
            rows.append(jnp.sum(pw[16:32], axis=0, keepdims=True))
        dec_ref[g] = jnp.concatenate(rows, axis=0)
        return prob

    prob0 = jnp.zeros((64, bb), jnp.float32)
    prob0 = jax.lax.fori_loop(0, 16, fwd_group_nostore, prob0)
    jax.lax.fori_loop(0, _NG - 16, fwd_group, prob0)

    def _step_back(state, w0, w1):
        w = jnp.where(state < 32, w0, w1)
        sh = state & 31
        d = (w >> sh) & 1
        return (state >> 1) + d * 32

    def tb_group(gi, state):
        tile = dec_ref[31 - gi]
        for i in range(7, -1, -1):
            state = _step_back(state, tile[2 * i : 2 * i + 1], tile[2 * i + 1 : 2 * i + 2])
        return state

    state = jnp.zeros((1, bb), jnp.int32)
    state = jax.lax.fori_loop(0, 16, tb_group, state)

    def tb_out_group(gi, state):
        g = 15 - gi
        tile = dec_ref[g]
        bits = [None] * 8
        for i in range(7, -1, -1):
            bits[i] = ((state + 1) & 1).astype(jnp.float32)
            state = _step_back(state, tile[2 * i : 2 * i + 1], tile[2 * i + 1 : 2 * i + 2])
        out_ref[g] = jnp.concatenate(bits, axis=0)
        return state

    jax.lax.fori_loop(0, 16, tb_out_group, state)


def kernel(x):
    b = x.shape[0]
    bb = min(_BB, b)
    x0 = jnp.transpose(x[:, 0::2]).reshape(16, 8, b)
    x1 = jnp.transpose(x[:, 1::2]).reshape(16, 8, b)
    res = pl.pallas_call(
        _viterbi_body,
        out_shape=jax.ShapeDtypeStruct((16, 8, b), jnp.float32),
        grid=(b // bb,),
        in_specs=[
            pl.BlockSpec((16, 8, bb), lambda i: (0, 0, i)),
            pl.BlockSpec((16, 8, bb), lambda i: (0, 0, i)),
            pl.BlockSpec((32, 1), lambda i: (0, 0)),
            pl.BlockSpec((32, 1), lambda i: (0, 0)),
            pl.BlockSpec((32, 1), lambda i: (0, 0)),
            pl.BlockSpec((32, 1), lambda i: (0, 0)),
        ],
        out_specs=pl.BlockSpec((16, 8, bb), lambda i: (0, 0, i)),
        scratch_shapes=[pltpu.VMEM((_NG - 16, 16, bb), jnp.int32)],
    )(x0, x1, jnp.asarray(_AE), jnp.asarray(_BE), jnp.asarray(_WE), jnp.asarray(_WO))
    return res.reshape(_DET_LENGTH, b).transpose(1, 0)
